# baseline (speedup 1.0000x reference)
import functools

import jax
import jax.numpy as jnp
from jax import lax
from jax.experimental import pallas as pl
from jax.experimental.pallas import tpu as pltpu
from jax.experimental.pallas import tpu_sc as plsc

N = 10000
E = 320000
D = 128
NPAD = 10240
NC = 2
NS = 16
NW = NC * NS
L = 16
EPT = E // NW
C = 80
NCH = EPT // C
SLICE = NPAD // NS

_mesh = plsc.VectorSubcoreMesh(core_axis_name="c", subcore_axis_name="s")


@functools.partial(
    pl.kernel,
    out_type=jax.ShapeDtypeStruct((NC, NPAD), jnp.float32),
    mesh=_mesh,
    scratch_types=[
        pltpu.VMEM((NCH, C), jnp.int32),
        pltpu.VMEM((SLICE,), jnp.float32),
        pltpu.VMEM((C,), jnp.float32),
        pltpu.VMEM_SHARED((NPAD,), jnp.float32),
    ],
)
def _deg_kernel(dst_hbm, out_hbm, idx_v, buf_v, ones_v, deg_sh):
    c = lax.axis_index("c")
    s = lax.axis_index("s")
    wid = c * NS + s
    pltpu.sync_copy(dst_hbm.at[wid], idx_v)
    zeros16 = jnp.zeros((L,), jnp.float32)
    ones16 = jnp.ones((L,), jnp.float32)

    def zero_body(i, carry):
        buf_v[pl.ds(i * L, L)] = zeros16
        return carry

    lax.fori_loop(0, SLICE // L, zero_body, 0)
    for q in range(C // L):
        ones_v[pl.ds(q * L, L)] = ones16
    pltpu.sync_copy(buf_v, deg_sh.at[pl.ds(s * SLICE, SLICE)])
    plsc.subcore_barrier()

    def body(j, carry):
        pltpu.sync_copy(ones_v, deg_sh.at[idx_v.at[j]], add=True)
        return carry

    lax.fori_loop(0, NCH, body, 0)
    plsc.subcore_barrier()
    pltpu.sync_copy(deg_sh.at[pl.ds(s * SLICE, SLICE)],
                    out_hbm.at[c, pl.ds(s * SLICE, SLICE)])


NB = 4
NI = 8


@functools.partial(
    pl.kernel,
    out_type=jax.ShapeDtypeStruct((NC, NPAD, D), jnp.float32),
    mesh=_mesh,
    scratch_types=[
        [pltpu.VMEM((C,), jnp.int32)] * NI,
        [pltpu.VMEM((C,), jnp.int32)] * NI,
        [pltpu.VMEM((C, D), jnp.float32)] * NB,
        pltpu.VMEM_SHARED((NPAD, D), jnp.float32),
        [pltpu.SemaphoreType.DMA] * NI,
        [pltpu.SemaphoreType.DMA] * NB,
        [pltpu.SemaphoreType.DMA] * NB,
    ],
)
def _scatter_kernel(g_hbm, src_hbm, dst_hbm, out_hbm, si, di, rbuf, accum_sh,
                    isem, gsem, asem):
    c = lax.axis_index("c")
    s = lax.axis_index("s")
    wid = c * NS + s

    zeros16 = jnp.zeros((L,), jnp.float32)

    def zero_body(i, carry):
        for q in range(D // L):
            rbuf[0][i, pl.ds(q * L, L)] = zeros16
        return carry

    lax.fori_loop(0, C, zero_body, 0)
    for k in range(SLICE // C):
        pltpu.sync_copy(rbuf[0], accum_sh.at[pl.ds(s * SLICE + k * C, C)])

    def li(j, m):
        pltpu.async_copy(src_hbm.at[wid, j], si[m], isem[m])
        pltpu.async_copy(dst_hbm.at[wid, j], di[m], isem[m])

    def wli(m):
        pltpu.make_async_copy(src_hbm.at[0, 0], si[m], isem[m]).wait()
        pltpu.make_async_copy(dst_hbm.at[0, 0], di[m], isem[m]).wait()

    def sg(j_islot, b):
        pltpu.async_copy(g_hbm.at[si[j_islot]], rbuf[b], gsem[b])

    def wg(b):
        pltpu.make_async_copy(g_hbm.at[si[0]], rbuf[b], gsem[b]).wait()

    def sa(j_islot, b):
        pltpu.async_copy(rbuf[b], accum_sh.at[di[j_islot]], asem[b],
                         add=True)

    def wa(b):
        pltpu.make_async_copy(rbuf[b], accum_sh.at[di[0]], asem[b]).wait()

    for m in range(NI):
        li(m, m)
    wli(0)
    wli(1)
    sg(0, 0)
    sg(1, 1)
    plsc.subcore_barrier()

    def step(j, t):
        b = t % NB
        wg(b)
        sa(t, b)
        b2 = (b + 2) % NB

        @pl.when(j + 2 < NCH)
        def _():
            @pl.when(j >= 2)
            def _():
                wa(b2)

                @pl.when(j + 6 < NCH)
                def _():
                    li(j + 6, (t - 2) % NI)

            wli((t + 2) % NI)
            sg((t + 2) % NI, b2)

    KU = (NCH // NI) * NI

    def body(k, carry):
        for t in range(NI):
            step(NI * k + t, t)
        return carry

    lax.fori_loop(0, KU // NI, body, 0)
    for j in range(KU, NCH):
        step(jnp.int32(j), j % NI)
    wa(1)
    wa(2)
    wa(3)
    wa(0)
    plsc.subcore_barrier()
    pltpu.sync_copy(accum_sh.at[pl.ds(s * SLICE, SLICE)],
                    out_hbm.at[c, pl.ds(s * SLICE, SLICE)])


@functools.partial(
    pl.kernel,
    out_type=jax.ShapeDtypeStruct((E, D), jnp.float32),
    mesh=_mesh,
    scratch_types=[
        pltpu.VMEM((NCH, C), jnp.int32),
        pltpu.VMEM((NCH, C), jnp.int32),
        pltpu.VMEM((C, D), jnp.float32),
        pltpu.VMEM((C, D), jnp.float32),
        pltpu.VMEM((C, D), jnp.float32),
        pltpu.VMEM((C, D), jnp.float32),
        pltpu.VMEM((C, D), jnp.float32),
        pltpu.VMEM((C, D), jnp.float32),
        pltpu.SemaphoreType.DMA,
        pltpu.SemaphoreType.DMA,
        pltpu.SemaphoreType.DMA,
        pltpu.SemaphoreType.DMA,
        pltpu.SemaphoreType.DMA,
        pltpu.SemaphoreType.DMA,
    ],
)
def _edge_kernel(ha_hbm, hb_hbm, src_hbm, dst_hbm, out_hbm, sidx_v, didx_v,
                 ra0_v, rb0_v, ro0_v, ra1_v, rb1_v, ro1_v,
                 sa0, sb0, so0, sa1, sb1, so1):
    c = lax.axis_index("c")
    s = lax.axis_index("s")
    wid = c * NS + s
    base = wid * EPT
    pltpu.sync_copy(src_hbm.at[wid], sidx_v)
    pltpu.sync_copy(dst_hbm.at[wid], didx_v)

    def sg(j, ra, rb, sga, sgb):
        pltpu.async_copy(ha_hbm.at[sidx_v.at[j]], ra, sga)
        pltpu.async_copy(hb_hbm.at[didx_v.at[j]], rb, sgb)

    def wg(ra, rb, sga, sgb):
        pltpu.make_async_copy(ha_hbm.at[sidx_v.at[0]], ra, sga).wait()
        pltpu.make_async_copy(hb_hbm.at[didx_v.at[0]], rb, sgb).wait()

    def comp(ra, rb, ro):
        def rbody(r, rc):
            for q in range(D // L):
                v = ra[r, pl.ds(q * L, L)] + rb[r, pl.ds(q * L, L)]
                ro[r, pl.ds(q * L, L)] = jnp.maximum(v, 0.0)
            return rc

        lax.fori_loop(0, C, rbody, 0)

    def so(j, ro, sem):
        pltpu.async_copy(ro, out_hbm.at[pl.ds(base + j * C, C)], sem)

    def wo(ro, sem):
        pltpu.make_async_copy(ro, out_hbm.at[pl.ds(base, C)], sem).wait()

    sg(0, ra0_v, rb0_v, sa0, sb0)
    sg(1, ra1_v, rb1_v, sa1, sb1)

    def body(k, carry):
        j0 = 2 * k
        wg(ra0_v, rb0_v, sa0, sb0)

        @pl.when(k > 0)
        def _():
            wo(ro0_v, so0)

        comp(ra0_v, rb0_v, ro0_v)

        @pl.when(j0 + 2 < NCH)
        def _():
            sg(j0 + 2, ra0_v, rb0_v, sa0, sb0)

        so(j0, ro0_v, so0)

        wg(ra1_v, rb1_v, sa1, sb1)

        @pl.when(k > 0)
        def _():
            wo(ro1_v, so1)

        comp(ra1_v, rb1_v, ro1_v)

        @pl.when(j0 + 3 < NCH)
        def _():
            sg(j0 + 3, ra1_v, rb1_v, sa1, sb1)

        so(j0 + 1, ro1_v, so1)
        return carry

    lax.fori_loop(0, (NCH - 1) // 2, body, 0)
    wg(ra0_v, rb0_v, sa0, sb0)
    comp(ra0_v, rb0_v, ro0_v)
    wo(ro0_v, so0)
    so(NCH - 1, ro0_v, so0)
    wo(ro0_v, so0)
    wo(ro1_v, so1)


RB = 1024


def _tc1a_body(x_ref, w_ref, xw_ref):
    xw_ref[...] = jnp.dot(x_ref[...], w_ref[...],
                          preferred_element_type=jnp.float32)


def _tc1a(x_pad, W1):
    return pl.pallas_call(
        _tc1a_body,
        grid=(NPAD // RB,),
        in_specs=[
            pl.BlockSpec((RB, D), lambda i: (i, 0)),
            pl.BlockSpec((D, D), lambda i: (0, 0)),
        ],
        out_specs=pl.BlockSpec((RB, D), lambda i: (i, 0)),
        out_shape=jax.ShapeDtypeStruct((NPAD, D), jnp.float32),
    )(x_pad, W1)


def _tc1b_body(degT_ref, xw_ref, g_ref, dis_ref):
    deg = jnp.sum(degT_ref[...], axis=1, keepdims=True) + 1.0
    dis = lax.rsqrt(deg)
    g_ref[...] = dis * xw_ref[...]
    dis_ref[...] = dis


def _tc1b(degT, xw):
    return pl.pallas_call(
        _tc1b_body,
        grid=(NPAD // RB,),
        in_specs=[
            pl.BlockSpec((RB, NC), lambda i: (i, 0)),
            pl.BlockSpec((RB, D), lambda i: (i, 0)),
        ],
        out_specs=[
            pl.BlockSpec((RB, D), lambda i: (i, 0)),
            pl.BlockSpec((RB, 1), lambda i: (i, 0)),
        ],
        out_shape=[
            jax.ShapeDtypeStruct((NPAD, D), jnp.float32),
            jax.ShapeDtypeStruct((NPAD, 1), jnp.float32),
        ],
    )(degT, xw)


def _tc2_body(sa_ref, sb_ref, g_ref, dis_ref, b_ref, w_ref, out_ref):
    dis = dis_ref[...]
    h = dis * (sa_ref[...] + sb_ref[...] + g_ref[...]) + b_ref[...]
    h = jnp.where(h >= 0, h, 0.1 * h)
    out_ref[...] = dis * jnp.dot(h, w_ref[...],
                                 preferred_element_type=jnp.float32)


def _tc2(sa, sb, g, dis, b, W2):
    return pl.pallas_call(
        _tc2_body,
        grid=(NPAD // RB,),
        in_specs=[
            pl.BlockSpec((RB, D), lambda i: (i, 0)),
            pl.BlockSpec((RB, D), lambda i: (i, 0)),
            pl.BlockSpec((RB, D), lambda i: (i, 0)),
            pl.BlockSpec((RB, 1), lambda i: (i, 0)),
            pl.BlockSpec((1, D), lambda i: (0, 0)),
            pl.BlockSpec((D, D), lambda i: (0, 0)),
        ],
        out_specs=pl.BlockSpec((RB, D), lambda i: (i, 0)),
        out_shape=jax.ShapeDtypeStruct((NPAD, D), jnp.float32),
    )(sa, sb, g, dis, b, W2)


def _tc3_body(sa_ref, sb_ref, g_ref, dis_ref, b_ref, wa_ref, wb_ref, bfc_ref,
              ha_ref, hb_ref):
    dis = dis_ref[...]
    h = dis * (sa_ref[...] + sb_ref[...] + g_ref[...]) + b_ref[...]
    h = jnp.where(h >= 0, h, 0.1 * h)
    ha_ref[...] = jnp.dot(h, wa_ref[...],
                          preferred_element_type=jnp.float32) + bfc_ref[...]
    hb_ref[...] = jnp.dot(h, wb_ref[...], preferred_element_type=jnp.float32)


def _tc3(sa, sb, g, dis, b, WfcA, WfcB, bfc):
    return pl.pallas_call(
        _tc3_body,
        grid=(NPAD // RB,),
        in_specs=[
            pl.BlockSpec((RB, D), lambda i: (i, 0)),
            pl.BlockSpec((RB, D), lambda i: (i, 0)),
            pl.BlockSpec((RB, D), lambda i: (i, 0)),
            pl.BlockSpec((RB, 1), lambda i: (i, 0)),
            pl.BlockSpec((1, D), lambda i: (0, 0)),
            pl.BlockSpec((D, D), lambda i: (0, 0)),
            pl.BlockSpec((D, D), lambda i: (0, 0)),
            pl.BlockSpec((1, D), lambda i: (0, 0)),
        ],
        out_specs=[
            pl.BlockSpec((RB, D), lambda i: (i, 0)),
            pl.BlockSpec((RB, D), lambda i: (i, 0)),
        ],
        out_shape=[
            jax.ShapeDtypeStruct((NPAD, D), jnp.float32),
            jax.ShapeDtypeStruct((NPAD, D), jnp.float32),
        ],
    )(sa, sb, g, dis, b, WfcA, WfcB, bfc)


def kernel(x, edge_index, W1, b1, W2, b2, Wfc, bfc):
    src = edge_index[0].astype(jnp.int32)
    dst = edge_index[1].astype(jnp.int32)
    src3 = src.reshape(NW, NCH, C)
    dst3 = dst.reshape(NW, NCH, C)
    x_pad = jnp.zeros((NPAD, D), jnp.float32).at[:N].set(x)

    xw = _tc1a(x_pad, W1)
    deg2 = _deg_kernel(dst3)
    g1, dis = _tc1b(deg2.T, xw)
    s1 = _scatter_kernel(g1, src3, dst3)
    g2 = _tc2(s1[0], s1[1], g1, dis, b1.reshape(1, D), W2)
    s2 = _scatter_kernel(g2, src3, dst3)
    ha, hb = _tc3(s2[0], s2[1], g2, dis, b2.reshape(1, D),
                  Wfc[:D], Wfc[D:], bfc.reshape(1, D))
    return _edge_kernel(ha, hb, src3, dst3)


# speedup vs baseline: 1.0212x; 1.0212x over previous
import functools

import jax
import jax.numpy as jnp
from jax import lax
from jax.experimental import pallas as pl
from jax.experimental.pallas import tpu as pltpu
from jax.experimental.pallas import tpu_sc as plsc

N = 10000
E = 320000
D = 128
NPAD = 10240
NC = 2
NS = 16
NW = NC * NS
L = 16
EPT = E // NW
C = 80
NCH = EPT // C
SLICE = NPAD // NS

_mesh = plsc.VectorSubcoreMesh(core_axis_name="c", subcore_axis_name="s")


@functools.partial(
    pl.kernel,
    out_type=jax.ShapeDtypeStruct((NC, NPAD), jnp.float32),
    mesh=_mesh,
    scratch_types=[
        pltpu.VMEM((NCH, C), jnp.int32),
        pltpu.VMEM((SLICE,), jnp.float32),
        pltpu.VMEM((C,), jnp.float32),
        pltpu.VMEM_SHARED((NPAD,), jnp.float32),
        pltpu.SemaphoreType.DMA,
    ],
)
def _deg_kernel(dst_hbm, out_hbm, idx_v, buf_v, ones_v, deg_sh, dsem):
    c = lax.axis_index("c")
    s = lax.axis_index("s")
    wid = c * NS + s
    pltpu.sync_copy(dst_hbm.at[wid], idx_v)
    zeros16 = jnp.zeros((L,), jnp.float32)
    ones16 = jnp.ones((L,), jnp.float32)

    def zero_body(i, carry):
        buf_v[pl.ds(i * L, L)] = zeros16
        return carry

    lax.fori_loop(0, SLICE // L, zero_body, 0)
    for q in range(C // L):
        ones_v[pl.ds(q * L, L)] = ones16
    pltpu.sync_copy(buf_v, deg_sh.at[pl.ds(s * SLICE, SLICE)])
    plsc.subcore_barrier()

    def body(j, carry):
        pltpu.async_copy(ones_v, deg_sh.at[idx_v.at[j]], dsem, add=True)
        return carry

    lax.fori_loop(0, NCH, body, 0)

    def drain(j, carry):
        pltpu.make_async_copy(ones_v, deg_sh.at[idx_v.at[0]], dsem).wait()
        return carry

    lax.fori_loop(0, NCH, drain, 0)
    plsc.subcore_barrier()
    pltpu.sync_copy(deg_sh.at[pl.ds(s * SLICE, SLICE)],
                    out_hbm.at[c, pl.ds(s * SLICE, SLICE)])


NB = 4
NI = 8


@functools.partial(
    pl.kernel,
    out_type=jax.ShapeDtypeStruct((NC, NPAD, D), jnp.float32),
    mesh=_mesh,
    scratch_types=[
        [pltpu.VMEM((C,), jnp.int32)] * NI,
        [pltpu.VMEM((C,), jnp.int32)] * NI,
        [pltpu.VMEM((C, D), jnp.float32)] * NB,
        pltpu.VMEM_SHARED((NPAD, D), jnp.float32),
        [pltpu.SemaphoreType.DMA] * NI,
        [pltpu.SemaphoreType.DMA] * NB,
        [pltpu.SemaphoreType.DMA] * NB,
    ],
)
def _scatter_kernel(g_hbm, src_hbm, dst_hbm, out_hbm, si, di, rbuf, accum_sh,
                    isem, gsem, asem):
    c = lax.axis_index("c")
    s = lax.axis_index("s")
    wid = c * NS + s

    zeros16 = jnp.zeros((L,), jnp.float32)

    def zero_body(i, carry):
        for q in range(D // L):
            rbuf[0][i, pl.ds(q * L, L)] = zeros16
        return carry

    lax.fori_loop(0, C, zero_body, 0)
    for k in range(SLICE // C):
        pltpu.sync_copy(rbuf[0], accum_sh.at[pl.ds(s * SLICE + k * C, C)])

    def li(j, m):
        pltpu.async_copy(src_hbm.at[wid, j], si[m], isem[m])
        pltpu.async_copy(dst_hbm.at[wid, j], di[m], isem[m])

    def wli(m):
        pltpu.make_async_copy(src_hbm.at[0, 0], si[m], isem[m]).wait()
        pltpu.make_async_copy(dst_hbm.at[0, 0], di[m], isem[m]).wait()

    def sg(j_islot, b):
        pltpu.async_copy(g_hbm.at[si[j_islot]], rbuf[b], gsem[b])

    def wg(b):
        pltpu.make_async_copy(g_hbm.at[si[0]], rbuf[b], gsem[b]).wait()

    def sa(j_islot, b):
        pltpu.async_copy(rbuf[b], accum_sh.at[di[j_islot]], asem[b],
                         add=True)

    def wa(b):
        pltpu.make_async_copy(rbuf[b], accum_sh.at[di[0]], asem[b]).wait()

    for m in range(NI):
        li(m, m)
    wli(0)
    wli(1)
    sg(0, 0)
    sg(1, 1)
    plsc.subcore_barrier()

    def step(j, t):
        b = t % NB
        wg(b)
        sa(t, b)
        b2 = (b + 2) % NB

        @pl.when(j + 2 < NCH)
        def _():
            @pl.when(j >= 2)
            def _():
                wa(b2)

                @pl.when(j + 6 < NCH)
                def _():
                    li(j + 6, (t - 2) % NI)

            wli((t + 2) % NI)
            sg((t + 2) % NI, b2)

    KU = (NCH // NI) * NI

    def body(k, carry):
        for t in range(NI):
            step(NI * k + t, t)
        return carry

    lax.fori_loop(0, KU // NI, body, 0)
    for j in range(KU, NCH):
        step(jnp.int32(j), j % NI)
    wa(1)
    wa(2)
    wa(3)
    wa(0)
    plsc.subcore_barrier()
    pltpu.sync_copy(accum_sh.at[pl.ds(s * SLICE, SLICE)],
                    out_hbm.at[c, pl.ds(s * SLICE, SLICE)])


@functools.partial(
    pl.kernel,
    out_type=jax.ShapeDtypeStruct((E, D), jnp.float32),
    mesh=_mesh,
    scratch_types=[
        pltpu.VMEM((NCH, C), jnp.int32),
        pltpu.VMEM((NCH, C), jnp.int32),
        pltpu.VMEM((C, D), jnp.float32),
        pltpu.VMEM((C, D), jnp.float32),
        pltpu.VMEM((C, D), jnp.float32),
        pltpu.VMEM((C, D), jnp.float32),
        pltpu.VMEM((C, D), jnp.float32),
        pltpu.VMEM((C, D), jnp.float32),
        pltpu.SemaphoreType.DMA,
        pltpu.SemaphoreType.DMA,
        pltpu.SemaphoreType.DMA,
        pltpu.SemaphoreType.DMA,
        pltpu.SemaphoreType.DMA,
        pltpu.SemaphoreType.DMA,
    ],
)
def _edge_kernel(ha_hbm, hb_hbm, src_hbm, dst_hbm, out_hbm, sidx_v, didx_v,
                 ra0_v, rb0_v, ro0_v, ra1_v, rb1_v, ro1_v,
                 sa0, sb0, so0, sa1, sb1, so1):
    c = lax.axis_index("c")
    s = lax.axis_index("s")
    wid = c * NS + s
    base = wid * EPT
    pltpu.sync_copy(src_hbm.at[wid], sidx_v)
    pltpu.sync_copy(dst_hbm.at[wid], didx_v)

    def sg(j, ra, rb, sga, sgb):
        pltpu.async_copy(ha_hbm.at[sidx_v.at[j]], ra, sga)
        pltpu.async_copy(hb_hbm.at[didx_v.at[j]], rb, sgb)

    def wg(ra, rb, sga, sgb):
        pltpu.make_async_copy(ha_hbm.at[sidx_v.at[0]], ra, sga).wait()
        pltpu.make_async_copy(hb_hbm.at[didx_v.at[0]], rb, sgb).wait()

    def comp(ra, rb, ro):
        def rbody(r, rc):
            for q in range(D // L):
                v = ra[r, pl.ds(q * L, L)] + rb[r, pl.ds(q * L, L)]
                ro[r, pl.ds(q * L, L)] = jnp.maximum(v, 0.0)
            return rc

        lax.fori_loop(0, C, rbody, 0)

    def so(j, ro, sem):
        pltpu.async_copy(ro, out_hbm.at[pl.ds(base + j * C, C)], sem)

    def wo(ro, sem):
        pltpu.make_async_copy(ro, out_hbm.at[pl.ds(base, C)], sem).wait()

    sg(0, ra0_v, rb0_v, sa0, sb0)
    sg(1, ra1_v, rb1_v, sa1, sb1)

    def body(k, carry):
        j0 = 2 * k
        wg(ra0_v, rb0_v, sa0, sb0)

        @pl.when(k > 0)
        def _():
            wo(ro0_v, so0)

        comp(ra0_v, rb0_v, ro0_v)

        @pl.when(j0 + 2 < NCH)
        def _():
            sg(j0 + 2, ra0_v, rb0_v, sa0, sb0)

        so(j0, ro0_v, so0)

        wg(ra1_v, rb1_v, sa1, sb1)

        @pl.when(k > 0)
        def _():
            wo(ro1_v, so1)

        comp(ra1_v, rb1_v, ro1_v)

        @pl.when(j0 + 3 < NCH)
        def _():
            sg(j0 + 3, ra1_v, rb1_v, sa1, sb1)

        so(j0 + 1, ro1_v, so1)
        return carry

    lax.fori_loop(0, (NCH - 1) // 2, body, 0)
    wg(ra0_v, rb0_v, sa0, sb0)
    comp(ra0_v, rb0_v, ro0_v)
    wo(ro0_v, so0)
    so(NCH - 1, ro0_v, so0)
    wo(ro0_v, so0)
    wo(ro1_v, so1)


RB = 1024


def _tc1_body(degT_ref, x_ref, w_ref, g_ref, dis_ref):
    deg = jnp.sum(degT_ref[...], axis=1, keepdims=True) + 1.0
    dis = lax.rsqrt(deg)
    g_ref[...] = dis * jnp.dot(x_ref[...], w_ref[...],
                               preferred_element_type=jnp.float32)
    dis_ref[...] = dis


def _tc1(degT, x_pad, W1):
    return pl.pallas_call(
        _tc1_body,
        grid=(NPAD // RB,),
        in_specs=[
            pl.BlockSpec((RB, NC), lambda i: (i, 0)),
            pl.BlockSpec((RB, D), lambda i: (i, 0)),
            pl.BlockSpec((D, D), lambda i: (0, 0)),
        ],
        out_specs=[
            pl.BlockSpec((RB, D), lambda i: (i, 0)),
            pl.BlockSpec((RB, 1), lambda i: (i, 0)),
        ],
        out_shape=[
            jax.ShapeDtypeStruct((NPAD, D), jnp.float32),
            jax.ShapeDtypeStruct((NPAD, 1), jnp.float32),
        ],
    )(degT, x_pad, W1)


def _tc2_body(sa_ref, sb_ref, g_ref, dis_ref, b_ref, w_ref, out_ref):
    dis = dis_ref[...]
    h = dis * (sa_ref[...] + sb_ref[...] + g_ref[...]) + b_ref[...]
    h = jnp.where(h >= 0, h, 0.1 * h)
    out_ref[...] = dis * jnp.dot(h, w_ref[...],
                                 preferred_element_type=jnp.float32)


def _tc2(sa, sb, g, dis, b, W2):
    return pl.pallas_call(
        _tc2_body,
        grid=(NPAD // RB,),
        in_specs=[
            pl.BlockSpec((RB, D), lambda i: (i, 0)),
            pl.BlockSpec((RB, D), lambda i: (i, 0)),
            pl.BlockSpec((RB, D), lambda i: (i, 0)),
            pl.BlockSpec((RB, 1), lambda i: (i, 0)),
            pl.BlockSpec((1, D), lambda i: (0, 0)),
            pl.BlockSpec((D, D), lambda i: (0, 0)),
        ],
        out_specs=pl.BlockSpec((RB, D), lambda i: (i, 0)),
        out_shape=jax.ShapeDtypeStruct((NPAD, D), jnp.float32),
    )(sa, sb, g, dis, b, W2)


def _tc3_body(sa_ref, sb_ref, g_ref, dis_ref, b_ref, wa_ref, wb_ref, bfc_ref,
              ha_ref, hb_ref):
    dis = dis_ref[...]
    h = dis * (sa_ref[...] + sb_ref[...] + g_ref[...]) + b_ref[...]
    h = jnp.where(h >= 0, h, 0.1 * h)
    ha_ref[...] = jnp.dot(h, wa_ref[...],
                          preferred_element_type=jnp.float32) + bfc_ref[...]
    hb_ref[...] = jnp.dot(h, wb_ref[...], preferred_element_type=jnp.float32)


def _tc3(sa, sb, g, dis, b, WfcA, WfcB, bfc):
    return pl.pallas_call(
        _tc3_body,
        grid=(NPAD // RB,),
        in_specs=[
            pl.BlockSpec((RB, D), lambda i: (i, 0)),
            pl.BlockSpec((RB, D), lambda i: (i, 0)),
            pl.BlockSpec((RB, D), lambda i: (i, 0)),
            pl.BlockSpec((RB, 1), lambda i: (i, 0)),
            pl.BlockSpec((1, D), lambda i: (0, 0)),
            pl.BlockSpec((D, D), lambda i: (0, 0)),
            pl.BlockSpec((D, D), lambda i: (0, 0)),
            pl.BlockSpec((1, D), lambda i: (0, 0)),
        ],
        out_specs=[
            pl.BlockSpec((RB, D), lambda i: (i, 0)),
            pl.BlockSpec((RB, D), lambda i: (i, 0)),
        ],
        out_shape=[
            jax.ShapeDtypeStruct((NPAD, D), jnp.float32),
            jax.ShapeDtypeStruct((NPAD, D), jnp.float32),
        ],
    )(sa, sb, g, dis, b, WfcA, WfcB, bfc)


def kernel(x, edge_index, W1, b1, W2, b2, Wfc, bfc):
    src = edge_index[0].astype(jnp.int32)
    dst = edge_index[1].astype(jnp.int32)
    src3 = src.reshape(NW, NCH, C)
    dst3 = dst.reshape(NW, NCH, C)
    x_pad = jnp.zeros((NPAD, D), jnp.float32).at[:N].set(x)

    deg2 = _deg_kernel(dst3)
    g1, dis = _tc1(deg2.T, x_pad, W1)
    s1 = _scatter_kernel(g1, src3, dst3)
    g2 = _tc2(s1[0], s1[1], g1, dis, b1.reshape(1, D), W2)
    s2 = _scatter_kernel(g2, src3, dst3)
    ha, hb = _tc3(s2[0], s2[1], g2, dis, b2.reshape(1, D),
                  Wfc[:D], Wfc[D:], bfc.reshape(1, D))
    return _edge_kernel(ha, hb, src3, dst3)
